# baseline (speedup 1.0000x reference)
import functools

import jax
import jax.numpy as jnp
from jax import lax
from jax.experimental import pallas as pl
from jax.experimental.pallas import tpu as pltpu
from jax.experimental.pallas import tpu_sc as plsc

E = 320000
N = 10000
D = 128
N_RBF = 8

NC = 2
NS = 16
NW = NC * NS
NHALF = 2
EH = E // NHALF
EPW = EH // NW
CH = 128
NL = EPW // CH + 1
FULL = (NL - 1) * CH
NTRASH = 8
TROWS = N + NTRASH
STRIPE = 640
ZR = 80


def _edge_product(e_rbf, m_half, W_edge, blk_off):
    BE = 8000

    def body(e_ref, m_ref, w_ref, o_ref):
        c = jnp.dot(e_ref[...], w_ref[...], preferred_element_type=jnp.float32)
        o_ref[...] = c * m_ref[...].astype(jnp.float32)

    return pl.pallas_call(
        body,
        grid=(EH // BE,),
        in_specs=[
            pl.BlockSpec((BE, N_RBF), lambda i: (i + blk_off, 0)),
            pl.BlockSpec((BE, D), lambda i: (i + blk_off, 0)),
            pl.BlockSpec((N_RBF, D), lambda i: (0, 0)),
        ],
        out_specs=pl.BlockSpec((BE, D), lambda i: (i, 0)),
        out_shape=jax.ShapeDtypeStruct((EH, D), jnp.float32),
    )(e_rbf, m_half, W_edge)


def _sc_segment_sum(prod, idx4, zrows):
    mesh = plsc.VectorSubcoreMesh(core_axis_name="c", subcore_axis_name="s")

    @functools.partial(
        pl.kernel,
        mesh=mesh,
        out_type=jax.ShapeDtypeStruct((NC, N, D), jnp.float32),
        scratch_types=[
            pltpu.VMEM((NL, CH), jnp.int32),
            pltpu.VMEM((2, CH, D), jnp.float32),
            pltpu.VMEM_SHARED((TROWS, D), jnp.float32),
            pltpu.SemaphoreType.DMA,
            pltpu.SemaphoreType.DMA,
            pltpu.SemaphoreType.DMA,
            pltpu.SemaphoreType.DMA,
        ],
    )
    def k(prod_hbm, idx_hbm, z_hbm, out_hbm, idx_v, bufs, table,
          lsem0, lsem1, ssem0, ssem1):
        c = lax.axis_index("c")
        s = lax.axis_index("s")
        nbase = s * STRIPE
        trips = jnp.where(s < NS - 1, STRIPE // ZR, (N - (NS - 1) * STRIPE) // ZR)
        zbuf = bufs.at[0, pl.ds(0, ZR)]

        pltpu.sync_copy(z_hbm, zbuf)

        def zero_step(i, _):
            pltpu.sync_copy(zbuf, table.at[pl.ds(nbase + i * ZR, ZR)])
            return _

        lax.fori_loop(0, trips, zero_step, 0)
        plsc.subcore_barrier()

        ebase = (c * NS + s) * EPW
        pltpu.sync_copy(idx_hbm.at[c, s], idx_v)

        lsems = (lsem0, lsem1)
        ssems = (ssem0, ssem1)
        lh = [None, None]
        sh = [None, None]
        lh[0] = pltpu.async_copy(prod_hbm.at[pl.ds(ebase, CH)], bufs.at[0], lsem0)
        for j in range(NL):
            cur = j % 2
            nxt = 1 - cur
            lh[cur].wait()
            if j >= 1:
                sh[nxt].wait()
            if j + 1 < NL:
                off = (j + 1) * CH if j + 1 < NL - 1 else EPW - CH
                lh[nxt] = pltpu.async_copy(
                    prod_hbm.at[pl.ds(ebase + off, CH)], bufs.at[nxt], lsems[nxt])
            sh[cur] = pltpu.async_copy(bufs.at[cur], table.at[idx_v.at[j]],
                                       ssems[cur], add=True)
        sh[(NL - 1) % 2].wait()
        plsc.subcore_barrier()

        def out_step(i, _):
            pltpu.sync_copy(table.at[pl.ds(nbase + i * ZR, ZR)], zbuf)
            pltpu.sync_copy(zbuf, out_hbm.at[c, pl.ds(nbase + i * ZR, ZR)])
            return _

        lax.fori_loop(0, trips, out_step, 0)

    return k(prod, idx4, zrows)


def _mlp(pa, pb, W1, b1, W2, b2, W3, b3, W_final):
    BN = 1000

    def swish(x):
        return x / (1.0 + jnp.exp(-x))

    def body(pa_ref, pb_ref, w1, b1r, w2, b2r, w3, b3r, wf, o_ref):
        x = (pa_ref[0] + pa_ref[1]) + (pb_ref[0] + pb_ref[1])
        x = swish(jnp.dot(x, w1[...], preferred_element_type=jnp.float32) + b1r[...])
        x = swish(jnp.dot(x, w2[...], preferred_element_type=jnp.float32) + b2r[...])
        x = swish(jnp.dot(x, w3[...], preferred_element_type=jnp.float32) + b3r[...])
        o_ref[...] = jnp.dot(x, wf[...], preferred_element_type=jnp.float32)

    wspec = pl.BlockSpec((D, D), lambda i: (0, 0))
    bspec = pl.BlockSpec((1, D), lambda i: (0, 0))
    return pl.pallas_call(
        body,
        grid=(N // BN,),
        in_specs=[
            pl.BlockSpec((NC, BN, D), lambda i: (0, i, 0)),
            pl.BlockSpec((NC, BN, D), lambda i: (0, i, 0)),
            wspec, bspec, wspec, bspec, wspec, bspec, wspec,
        ],
        out_specs=pl.BlockSpec((BN, D), lambda i: (i, 0)),
        out_shape=jax.ShapeDtypeStruct((N, D), jnp.float32),
    )(pa, pb, W1, b1.reshape(1, D), W2, b2.reshape(1, D),
      W3, b3.reshape(1, D), W_final)


def _build_idx(idx_w):
    trash = (N + jnp.arange(NW, dtype=jnp.int32) % NTRASH)[:, None]
    head = idx_w[:, :FULL].reshape(NW, NL - 1, CH)
    tail = jnp.concatenate(
        [jnp.broadcast_to(trash, (NW, CH - (EPW - FULL))), idx_w[:, FULL:]],
        axis=1)[:, None, :]
    return jnp.concatenate([head, tail], axis=1).reshape(NC, NS, NL, CH)


def kernel(m_ji, e_rbf, nbr_list, num_atoms, W_edge, W1, b1, W2, b2, W3, b3, W_final):
    idx_h = nbr_list[:, 0].reshape(NHALF, NW, EPW)
    zrows = jnp.zeros((ZR, D), jnp.float32)
    m16 = m_ji.astype(jnp.bfloat16)
    partials = []
    for h in range(NHALF):
        prod_h = _edge_product(e_rbf, m16, W_edge, h * (EH // 8000))
        partials.append(_sc_segment_sum(prod_h, _build_idx(idx_h[h]), zrows))
    return _mlp(partials[0], partials[1], W1, b1, W2, b2, W3, b3, W_final)


# speedup vs baseline: 1.1017x; 1.1017x over previous
import functools

import jax
import jax.numpy as jnp
from jax import lax
from jax.experimental import pallas as pl
from jax.experimental.pallas import tpu as pltpu
from jax.experimental.pallas import tpu_sc as plsc

E = 320000
N = 10000
D = 128
N_RBF = 8

NC = 2
NS = 16
NW = NC * NS
NHALF = 2
EH = E // NHALF
EPW = EH // NW
CH = 96
NL = EPW // CH + 1
FULL = (NL - 1) * CH
NTRASH = 8
TROWS = N + NTRASH
STRIPE = 640
ZR = 80


def _edge_product(e_rbf, m_half, W_edge, blk_off):
    BE = 8000

    def body(e_ref, m_ref, w_ref, o_ref):
        c = jnp.dot(e_ref[...], w_ref[...], preferred_element_type=jnp.float32)
        o_ref[...] = c * m_ref[...]

    return pl.pallas_call(
        body,
        grid=(EH // BE,),
        in_specs=[
            pl.BlockSpec((BE, N_RBF), lambda i: (i + blk_off, 0)),
            pl.BlockSpec((BE, D), lambda i: (i + blk_off, 0)),
            pl.BlockSpec((N_RBF, D), lambda i: (0, 0)),
        ],
        out_specs=pl.BlockSpec((BE, D), lambda i: (i, 0)),
        out_shape=jax.ShapeDtypeStruct((EH, D), jnp.float32),
    )(e_rbf, m_half, W_edge)


def _sc_segment_sum(prod, idx4, zrows):
    mesh = plsc.VectorSubcoreMesh(core_axis_name="c", subcore_axis_name="s")

    @functools.partial(
        pl.kernel,
        mesh=mesh,
        out_type=jax.ShapeDtypeStruct((NC, N, D), jnp.float32),
        scratch_types=[
            pltpu.VMEM((NL, CH), jnp.int32),
            pltpu.VMEM((3, CH, D), jnp.float32),
            pltpu.VMEM_SHARED((TROWS, D), jnp.float32),
            pltpu.SemaphoreType.DMA,
            pltpu.SemaphoreType.DMA,
            pltpu.SemaphoreType.DMA,
            pltpu.SemaphoreType.DMA,
            pltpu.SemaphoreType.DMA,
            pltpu.SemaphoreType.DMA,
        ],
    )
    def k(prod_hbm, idx_hbm, z_hbm, out_hbm, idx_v, bufs, table,
          lsem0, lsem1, lsem2, ssem0, ssem1, ssem2):
        c = lax.axis_index("c")
        s = lax.axis_index("s")
        nbase = s * STRIPE
        trips = jnp.where(s < NS - 1, STRIPE // ZR, (N - (NS - 1) * STRIPE) // ZR)
        zbuf = bufs.at[0, pl.ds(0, ZR)]

        pltpu.sync_copy(z_hbm, zbuf)

        def zero_step(i, _):
            pltpu.sync_copy(zbuf, table.at[pl.ds(nbase + i * ZR, ZR)])
            return _

        lax.fori_loop(0, trips, zero_step, 0)
        plsc.subcore_barrier()

        ebase = (c * NS + s) * EPW
        pltpu.sync_copy(idx_hbm.at[c, s], idx_v)

        lsems = (lsem0, lsem1, lsem2)
        ssems = (ssem0, ssem1, ssem2)
        lh = [None, None, None]
        sh = [None, None, None]
        lh[0] = pltpu.async_copy(prod_hbm.at[pl.ds(ebase, CH)], bufs.at[0], lsem0)
        for j in range(NL):
            cur = j % 3
            nxt = (j + 1) % 3
            lh[cur].wait()
            if j >= 2:
                sh[nxt].wait()
            if j + 1 < NL:
                off = (j + 1) * CH if j + 1 < NL - 1 else EPW - CH
                lh[nxt] = pltpu.async_copy(
                    prod_hbm.at[pl.ds(ebase + off, CH)], bufs.at[nxt], lsems[nxt])
            sh[cur] = pltpu.async_copy(bufs.at[cur], table.at[idx_v.at[j]],
                                       ssems[cur], add=True)
        sh[(NL - 1) % 3].wait()
        sh[(NL - 2) % 3].wait()
        plsc.subcore_barrier()

        def out_step(i, _):
            pltpu.sync_copy(table.at[pl.ds(nbase + i * ZR, ZR)], zbuf)
            pltpu.sync_copy(zbuf, out_hbm.at[c, pl.ds(nbase + i * ZR, ZR)])
            return _

        lax.fori_loop(0, trips, out_step, 0)

    return k(prod, idx4, zrows)


def _mlp(pa, pb, W1, b1, W2, b2, W3, b3, W_final):
    BN = 1000

    def swish(x):
        return x / (1.0 + jnp.exp(-x))

    def body(pa_ref, pb_ref, w1, b1r, w2, b2r, w3, b3r, wf, o_ref):
        x = (pa_ref[0] + pa_ref[1]) + (pb_ref[0] + pb_ref[1])
        x = swish(jnp.dot(x, w1[...], preferred_element_type=jnp.float32) + b1r[...])
        x = swish(jnp.dot(x, w2[...], preferred_element_type=jnp.float32) + b2r[...])
        x = swish(jnp.dot(x, w3[...], preferred_element_type=jnp.float32) + b3r[...])
        o_ref[...] = jnp.dot(x, wf[...], preferred_element_type=jnp.float32)

    wspec = pl.BlockSpec((D, D), lambda i: (0, 0))
    bspec = pl.BlockSpec((1, D), lambda i: (0, 0))
    return pl.pallas_call(
        body,
        grid=(N // BN,),
        in_specs=[
            pl.BlockSpec((NC, BN, D), lambda i: (0, i, 0)),
            pl.BlockSpec((NC, BN, D), lambda i: (0, i, 0)),
            wspec, bspec, wspec, bspec, wspec, bspec, wspec,
        ],
        out_specs=pl.BlockSpec((BN, D), lambda i: (i, 0)),
        out_shape=jax.ShapeDtypeStruct((N, D), jnp.float32),
    )(pa, pb, W1, b1.reshape(1, D), W2, b2.reshape(1, D),
      W3, b3.reshape(1, D), W_final)


def _build_idx(idx_w):
    trash = (N + jnp.arange(NW, dtype=jnp.int32) % NTRASH)[:, None]
    head = idx_w[:, :FULL].reshape(NW, NL - 1, CH)
    tail = jnp.concatenate(
        [jnp.broadcast_to(trash, (NW, CH - (EPW - FULL))), idx_w[:, FULL:]],
        axis=1)[:, None, :]
    return jnp.concatenate([head, tail], axis=1).reshape(NC, NS, NL, CH)


def kernel(m_ji, e_rbf, nbr_list, num_atoms, W_edge, W1, b1, W2, b2, W3, b3, W_final):
    idx_h = nbr_list[:, 0].reshape(NHALF, NW, EPW)
    zrows = jnp.zeros((ZR, D), jnp.float32)
    partials = []
    for h in range(NHALF):
        prod_h = _edge_product(e_rbf, m_ji, W_edge, h * (EH // 8000))
        partials.append(_sc_segment_sum(prod_h, _build_idx(idx_h[h]), zrows))
    return _mlp(partials[0], partials[1], W1, b1, W2, b2, W3, b3, W_final)


# speedup vs baseline: 1.1519x; 1.0456x over previous
import functools

import jax
import jax.numpy as jnp
from jax import lax
from jax.experimental import pallas as pl
from jax.experimental.pallas import tpu as pltpu
from jax.experimental.pallas import tpu_sc as plsc

E = 320000
N = 10000
D = 128
N_RBF = 8

NC = 2
NS = 16
NW = NC * NS
NHALF = 1
EH = E // NHALF
EPW = EH // NW
CH = 128
NL = EPW // CH + 1
FULL = (NL - 1) * CH
NTRASH = 8
TROWS = N + NTRASH
STRIPE = 640
ZR = 80


def _edge_product(e_rbf, m_half, W_edge, blk_off):
    BE = 8000

    def body(e_ref, m_ref, w_ref, o_ref):
        c = jnp.dot(e_ref[...], w_ref[...], preferred_element_type=jnp.float32)
        o_ref[...] = c * m_ref[...]

    return pl.pallas_call(
        body,
        grid=(EH // BE,),
        in_specs=[
            pl.BlockSpec((BE, N_RBF), lambda i: (i + blk_off, 0)),
            pl.BlockSpec((BE, D), lambda i: (i + blk_off, 0)),
            pl.BlockSpec((N_RBF, D), lambda i: (0, 0)),
        ],
        out_specs=pl.BlockSpec((BE, D), lambda i: (i, 0)),
        out_shape=jax.ShapeDtypeStruct((EH, D), jnp.float32),
    )(e_rbf, m_half, W_edge)


def _sc_segment_sum(prod, idx4, zrows):
    mesh = plsc.VectorSubcoreMesh(core_axis_name="c", subcore_axis_name="s")

    @functools.partial(
        pl.kernel,
        mesh=mesh,
        out_type=jax.ShapeDtypeStruct((NC, N, D), jnp.float32),
        scratch_types=[
            pltpu.VMEM((NL, CH), jnp.int32),
            pltpu.VMEM((2, CH, D), jnp.float32),
            pltpu.VMEM_SHARED((TROWS, D), jnp.float32),
            pltpu.SemaphoreType.DMA,
            pltpu.SemaphoreType.DMA,
            pltpu.SemaphoreType.DMA,
            pltpu.SemaphoreType.DMA,
        ],
    )
    def k(prod_hbm, idx_hbm, z_hbm, out_hbm, idx_v, bufs, table,
          lsem0, lsem1, ssem0, ssem1):
        c = lax.axis_index("c")
        s = lax.axis_index("s")
        nbase = s * STRIPE
        trips = jnp.where(s < NS - 1, STRIPE // ZR, (N - (NS - 1) * STRIPE) // ZR)
        zbuf = bufs.at[0, pl.ds(0, ZR)]

        pltpu.sync_copy(z_hbm, zbuf)

        def zero_step(i, _):
            pltpu.sync_copy(zbuf, table.at[pl.ds(nbase + i * ZR, ZR)])
            return _

        lax.fori_loop(0, trips, zero_step, 0)
        plsc.subcore_barrier()

        ebase = (c * NS + s) * EPW
        pltpu.sync_copy(idx_hbm.at[c, s], idx_v)

        lsems = (lsem0, lsem1)
        ssems = (ssem0, ssem1)
        lh = [None, None]
        sh = [None, None]
        lh[0] = pltpu.async_copy(prod_hbm.at[pl.ds(ebase, CH)], bufs.at[0], lsem0)
        for j in range(NL):
            cur = j % 2
            nxt = 1 - cur
            lh[cur].wait()
            if j >= 1:
                sh[nxt].wait()
            if j + 1 < NL:
                off = (j + 1) * CH if j + 1 < NL - 1 else EPW - CH
                lh[nxt] = pltpu.async_copy(
                    prod_hbm.at[pl.ds(ebase + off, CH)], bufs.at[nxt], lsems[nxt])
            sh[cur] = pltpu.async_copy(bufs.at[cur], table.at[idx_v.at[j]],
                                       ssems[cur], add=True)
        sh[(NL - 1) % 2].wait()
        plsc.subcore_barrier()

        def out_step(i, _):
            pltpu.sync_copy(table.at[pl.ds(nbase + i * ZR, ZR)], zbuf)
            pltpu.sync_copy(zbuf, out_hbm.at[c, pl.ds(nbase + i * ZR, ZR)])
            return _

        lax.fori_loop(0, trips, out_step, 0)

    return k(prod, idx4, zrows)


def _mlp(pa, W1, b1, W2, b2, W3, b3, W_final):
    BN = 1000

    def swish(x):
        return x / (1.0 + jnp.exp(-x))

    def body(pa_ref, w1, b1r, w2, b2r, w3, b3r, wf, o_ref):
        x = pa_ref[0] + pa_ref[1]
        x = swish(jnp.dot(x, w1[...], preferred_element_type=jnp.float32) + b1r[...])
        x = swish(jnp.dot(x, w2[...], preferred_element_type=jnp.float32) + b2r[...])
        x = swish(jnp.dot(x, w3[...], preferred_element_type=jnp.float32) + b3r[...])
        o_ref[...] = jnp.dot(x, wf[...], preferred_element_type=jnp.float32)

    wspec = pl.BlockSpec((D, D), lambda i: (0, 0))
    bspec = pl.BlockSpec((1, D), lambda i: (0, 0))
    return pl.pallas_call(
        body,
        grid=(N // BN,),
        in_specs=[
            pl.BlockSpec((NC, BN, D), lambda i: (0, i, 0)),
            wspec, bspec, wspec, bspec, wspec, bspec, wspec,
        ],
        out_specs=pl.BlockSpec((BN, D), lambda i: (i, 0)),
        out_shape=jax.ShapeDtypeStruct((N, D), jnp.float32),
    )(pa, W1, b1.reshape(1, D), W2, b2.reshape(1, D),
      W3, b3.reshape(1, D), W_final)


def _build_idx(idx_w):
    trash = (N + jnp.arange(NW, dtype=jnp.int32) % NTRASH)[:, None]
    head = idx_w[:, :FULL].reshape(NW, NL - 1, CH)
    tail = jnp.concatenate(
        [jnp.broadcast_to(trash, (NW, CH - (EPW - FULL))), idx_w[:, FULL:]],
        axis=1)[:, None, :]
    return jnp.concatenate([head, tail], axis=1).reshape(NC, NS, NL, CH)


def kernel(m_ji, e_rbf, nbr_list, num_atoms, W_edge, W1, b1, W2, b2, W3, b3, W_final):
    idx_w = nbr_list[:, 0].reshape(NW, EPW)
    zrows = jnp.zeros((ZR, D), jnp.float32)
    prod = _edge_product(e_rbf, m_ji, W_edge, 0)
    partials = _sc_segment_sum(prod, _build_idx(idx_w), zrows)
    return _mlp(partials, W1, b1, W2, b2, W3, b3, W_final)
